# baseline (speedup 1.0000x reference)
import functools
import math

import jax
import jax.numpy as jnp
from jax import lax
from jax.experimental import pallas as pl
from jax.experimental.pallas import tpu as pltpu
from jax.experimental.pallas import tpu_sc as plsc

D_MODEL = 64
SCALE = 8.0
LANES = 16
G = 128


def _make_sc_gather(B, V):
    info = plsc.get_sparse_core_info()
    NC, NS = info.num_cores, info.num_subcores
    NW = NC * NS
    chunks = B // G
    per_w = chunks // NW

    mesh = plsc.VectorSubcoreMesh(core_axis_name="c", subcore_axis_name="s")

    @functools.partial(
        pl.kernel,
        out_type=jax.ShapeDtypeStruct((B, D_MODEL), jnp.float32),
        mesh=mesh,
        scratch_types=[
            pltpu.VMEM((G,), jnp.int32),
            pltpu.VMEM((G, D_MODEL), jnp.float32),
            pltpu.SemaphoreType.DMA,
        ],
        compiler_params=pltpu.CompilerParams(use_tc_tiling_on_sc=False),
    )
    def body(table_hbm, idx_hbm, out_hbm, idx_v, rows_v, sem):
        wid = lax.axis_index("s") * NC + lax.axis_index("c")

        def step(c, carry):
            r = wid * per_w + c
            pltpu.sync_copy(idx_hbm.at[r], idx_v)
            pltpu.async_copy(table_hbm.at[idx_v], rows_v, sem).wait()

            def scale_row(i, carry2):
                for j in range(D_MODEL // LANES):
                    sl = pl.ds(j * LANES, LANES)
                    rows_v[i, sl] = rows_v[i, sl] * SCALE
                return carry2

            lax.fori_loop(0, G, scale_row, 0)
            pltpu.sync_copy(rows_v, out_hbm.at[pl.ds(r * G, G)])
            return carry

        lax.fori_loop(0, per_w, step, 0)

    return body


def kernel(token_ids, embedding_weights):
    BATCH, HIST = token_ids.shape
    B = BATCH * HIST
    V = embedding_weights.shape[0]
    idx = token_ids.reshape(B // G, G)
    out = _make_sc_gather(B, V)(embedding_weights, idx)
    return out.reshape(BATCH, HIST, D_MODEL)


# speedup vs baseline: 1.2773x; 1.2773x over previous
import functools
import math

import jax
import jax.numpy as jnp
from jax import lax
from jax.experimental import pallas as pl
from jax.experimental.pallas import tpu as pltpu
from jax.experimental.pallas import tpu_sc as plsc

D_MODEL = 64
SCALE = 8.0
LANES = 16
G = 128
S = 2
CH = G * S
NB = 4


def _make_sc_gather(B, V):
    info = plsc.get_sparse_core_info()
    NC, NS = info.num_cores, info.num_subcores
    NW = NC * NS
    per_w = B // NW
    n_idx = per_w // G
    nch = per_w // CH

    mesh = plsc.VectorSubcoreMesh(core_axis_name="c", subcore_axis_name="s")

    @functools.partial(
        pl.kernel,
        out_type=jax.ShapeDtypeStruct((B, D_MODEL), jnp.float32),
        mesh=mesh,
        scratch_types=[
            pltpu.VMEM((n_idx, G), jnp.int32),
            [pltpu.VMEM((CH, D_MODEL), jnp.float32) for _ in range(NB)],
            [pltpu.SemaphoreType.DMA for _ in range(NB)],
            [pltpu.SemaphoreType.DMA for _ in range(NB)],
        ],
        compiler_params=pltpu.CompilerParams(use_tc_tiling_on_sc=False),
    )
    def body(table_hbm, idx_hbm, out_hbm, idx_all, bufs, gsems, ssems):
        wid = lax.axis_index("s") * NC + lax.axis_index("c")
        wrow = wid * per_w

        pltpu.sync_copy(idx_hbm.at[pl.ds(wid * n_idx, n_idx)], idx_all)

        def gather_descs(c, b):
            return [
                pltpu.make_async_copy(
                    table_hbm.at[idx_all.at[c * S + j]],
                    bufs[b].at[pl.ds(j * G, G)],
                    gsems[b],
                )
                for j in range(S)
            ]

        def scatter_desc(c, b):
            return pltpu.make_async_copy(
                bufs[b], out_hbm.at[pl.ds(wrow + c * CH, CH)], ssems[b]
            )

        def pre(c, b, waits_scatter):
            if waits_scatter:
                scatter_desc(c - NB, b).wait()
            for d in gather_descs(c, b):
                d.start()

        def post(c, b):
            for d in gather_descs(c, b):
                d.wait()
            buf = bufs[b]

            @plsc.parallel_loop(0, CH, unroll=4)
            def _scale(i):
                for j in range(D_MODEL // LANES):
                    sl = pl.ds(j * LANES, LANES)
                    buf[i, sl] = buf[i, sl] * SCALE

            scatter_desc(c, b).start()

        pre(0, 0, False)
        pre(1, 1, False)
        pre(2, 2, False)
        post(0, 0)
        pre(3, 3, False)
        post(1, 1)

        def round_body(k, carry):
            c0 = k * NB
            for b in range(NB):
                pre(c0 + b, b, True)
                post(c0 + b - 2, (b - 2) % NB)
            return carry

        lax.fori_loop(1, nch // NB, round_body, 0)

        post(nch - 2, (nch - 2) % NB)
        post(nch - 1, (nch - 1) % NB)
        for b in range(NB):
            scatter_desc(nch - NB + b, (nch - NB + b) % NB).wait()

    return body


def kernel(token_ids, embedding_weights):
    BATCH, HIST = token_ids.shape
    B = BATCH * HIST
    V = embedding_weights.shape[0]
    idx = token_ids.reshape(B // G, G)
    out = _make_sc_gather(B, V)(embedding_weights, idx)
    return out.reshape(BATCH, HIST, D_MODEL)
